# baseline (speedup 1.0000x reference)
import jax
import jax.numpy as jnp
from jax.experimental import pallas as pl

NUM_CLASSES_ = 1000
N_ = 16384
BLOCK_ROWS = 2048
BLOCK_COLS = 128


def _onehot_block(x_ref, o_ref):
    j = pl.program_id(1)
    xb = x_ref[0, 0, :]
    col = jax.lax.broadcasted_iota(jnp.int32, (BLOCK_ROWS, BLOCK_COLS), 1)
    col = col + j * BLOCK_COLS
    o_ref[:, :] = (xb[:, None] == col).astype(jnp.float32)


def kernel(x):
    nb = N_ // BLOCK_ROWS
    nc = pl.cdiv(NUM_CLASSES_, BLOCK_COLS)
    x3 = x.astype(jnp.int32).reshape(nb, 1, BLOCK_ROWS)
    out = pl.pallas_call(
        _onehot_block,
        grid=(nb, nc),
        in_specs=[pl.BlockSpec((1, 1, BLOCK_ROWS), lambda i, j: (i, 0, 0))],
        out_specs=pl.BlockSpec((BLOCK_ROWS, BLOCK_COLS), lambda i, j: (i, j)),
        out_shape=jax.ShapeDtypeStruct((N_, NUM_CLASSES_), jnp.float32),
    )(x3)
    return out
